# Initial kernel scaffold; baseline (speedup 1.0000x reference)
import jax
import jax.numpy as jnp
from jax.experimental import pallas as pl


def kernel(x, edge_index, edge_curvature, params):
    raise NotImplementedError("write your pallas kernel here")


# speedup vs baseline: 8.0019x; 8.0019x over previous
import functools

import jax
import jax.numpy as jnp
from jax import lax
from jax.experimental import pallas as pl
from jax.experimental.pallas import tpu as pltpu
from jax.experimental.pallas import tpu_sc as plsc

NN = 10000
EE = 320000
HD = 128
NCORE = 2
NSUB = 16
NW = NCORE * NSUB
CH = 64
BLK = 8
CHUNKS = 160
BLOCKS = CHUNKS // BLK
EPT = CHUNKS * CH
EPAD = EPT * NW
NPAD = 10240
DR = NPAD // HD
DRP = 128
DRT = DRP // NSUB
NPT = NPAD // NSUB


def _sc_body(mtab, avt, srcs, dsts, masks, agg_out, den_out,
             src_b, dst_b, msk_b, as_v, ad_v, den_v, exc_v, rows_v, idx_v,
             agg_sh, den_sh, sem):
    c = lax.axis_index("c")
    s = lax.axis_index("s")
    wid = c * NSUB + s

    pltpu.sync_copy(avt.at[0], as_v)
    pltpu.sync_copy(avt.at[1], ad_v)

    zeros16 = jnp.zeros((16,), jnp.float32)

    def zden(i, carry):
        for k in range(HD // 16):
            den_v[i, pl.ds(k * 16, 16)] = zeros16
        return carry

    lax.fori_loop(0, DR, zden, 0)

    def zrow(i, carry):
        for k in range(HD // 16):
            rows_v[i, pl.ds(k * 16, 16)] = zeros16
        return carry

    lax.fori_loop(0, CH, zrow, 0)

    for g in range(DR // 16):
        idx_v[0, pl.ds(g * 16, 16)] = (
            lax.iota(jnp.int32, 16) + jnp.full((16,), g * 16, jnp.int32))

    for k in range(NPT // CH):
        pltpu.sync_copy(rows_v, agg_sh.at[pl.ds(s * NPT + k * CH, CH)])
    pltpu.sync_copy(rows_v.at[pl.ds(0, DRT)], den_sh.at[pl.ds(s * DRT, DRT)])
    plsc.subcore_barrier()

    def block(b, carry):
        pltpu.sync_copy(srcs.at[wid, pl.ds(b * BLK, BLK)], src_b)
        pltpu.sync_copy(dsts.at[wid, pl.ds(b * BLK, BLK)], dst_b)
        pltpu.sync_copy(masks.at[wid, pl.ds(b * BLK, BLK)], msk_b)
        for q in range(BLK):
            for j in range(CH // 16):
                sid = src_b[q, pl.ds(j * 16, 16)]
                did = dst_b[q, pl.ds(j * 16, 16)]
                a = (plsc.load_gather(as_v, [sid])
                     + plsc.load_gather(ad_v, [did]))
                a = jnp.maximum(a, a * 0.2)
                a = jnp.minimum(jnp.maximum(a, -30.0), 30.0)
                ex = msk_b[q, pl.ds(j * 16, 16)] * jnp.exp(a)
                exc_v[pl.ds(j * 16, 16)] = ex
                plsc.addupdate_scatter(
                    den_v,
                    [lax.shift_right_logical(did, 7),
                     lax.bitwise_and(did, jnp.full((16,), 127, jnp.int32))],
                    ex)
            pltpu.async_copy(mtab.at[src_b.at[q]], rows_v, sem).wait()

            def scale_i(i, carry2):
                cb = plsc.load_gather(exc_v, [jnp.full((16,), i, jnp.int32)])
                for k in range(HD // 16):
                    rows_v[i, pl.ds(k * 16, 16)] = (
                        rows_v[i, pl.ds(k * 16, 16)] * cb)
                return carry2

            lax.fori_loop(0, CH, scale_i, 0)
            pltpu.sync_copy(rows_v, agg_sh.at[dst_b.at[q]], add=True)
        return carry

    lax.fori_loop(0, BLOCKS, block, 0)

    pltpu.sync_copy(den_v, den_sh.at[idx_v.at[0]], add=True)
    plsc.subcore_barrier()

    pltpu.sync_copy(agg_sh.at[pl.ds(s * NPT, NPT)],
                    agg_out.at[pl.ds(c * NPAD + s * NPT, NPT)])
    pltpu.sync_copy(den_sh.at[pl.ds(s * DRT, DRT)],
                    den_out.at[c, pl.ds(s * DRT, DRT)])


_sc_edge = functools.partial(
    pl.kernel,
    out_type=(
        jax.ShapeDtypeStruct((NCORE * NPAD, HD), jnp.float32),
        jax.ShapeDtypeStruct((NCORE, DRP, HD), jnp.float32),
    ),
    mesh=plsc.VectorSubcoreMesh(core_axis_name="c", subcore_axis_name="s",
                                num_cores=NCORE, num_subcores=NSUB),
    scratch_types=[
        pltpu.VMEM((BLK, CH), jnp.int32),
        pltpu.VMEM((BLK, CH), jnp.int32),
        pltpu.VMEM((BLK, CH), jnp.float32),
        pltpu.VMEM((NPAD,), jnp.float32),
        pltpu.VMEM((NPAD,), jnp.float32),
        pltpu.VMEM((DR, HD), jnp.float32),
        pltpu.VMEM((CH,), jnp.float32),
        pltpu.VMEM((CH, HD), jnp.float32),
        pltpu.VMEM((1, DR), jnp.int32),
        pltpu.VMEM_SHARED((NPAD, HD), jnp.float32),
        pltpu.VMEM_SHARED((DRP, HD), jnp.float32),
        pltpu.SemaphoreType.DMA,
    ],
    compiler_params=pltpu.CompilerParams(needs_layout_passes=False),
)(_sc_body)


def _tc_in_body(x_ref, w_ref, b_ref, out_ref):
    h = jnp.dot(x_ref[...], w_ref[...], preferred_element_type=jnp.float32)
    out_ref[...] = jnp.maximum(h + b_ref[...], 0.0)


def _tc_in(x, w, b):
    return pl.pallas_call(
        _tc_in_body,
        out_shape=jax.ShapeDtypeStruct((NN, HD), jnp.float32),
    )(x, w, b)


def _tc_pre_body(h_ref, w_ref, wself_ref, b_ref, a2_ref,
                 m_ref, aa_ref, hs_ref):
    h = h_ref[...]
    m = jnp.dot(h, w_ref[...], preferred_element_type=jnp.float32)
    m_ref[...] = m
    aa_ref[...] = jnp.dot(m, a2_ref[...], preferred_element_type=jnp.float32)
    hs_ref[...] = (jnp.dot(h, wself_ref[...],
                           preferred_element_type=jnp.float32) + b_ref[...])


def _tc_pre(h, w, wself, b, a2):
    return pl.pallas_call(
        _tc_pre_body,
        out_shape=(
            jax.ShapeDtypeStruct((NN, HD), jnp.float32),
            jax.ShapeDtypeStruct((NN, 2), jnp.float32),
            jax.ShapeDtypeStruct((NN, HD), jnp.float32),
        ),
    )(h, w, wself, b, a2)


def _tc_post_body(agg2_ref, den3_ref, hs_ref, g_ref, b_ref, out_ref):
    agg = agg2_ref[pl.ds(0, NN), :] + agg2_ref[pl.ds(NPAD, NN), :]
    den = (den3_ref[0] + den3_ref[1])[:NN, :]
    hv = agg / (den + 1e-16) + hs_ref[...]
    mu = jnp.mean(hv, axis=0)
    var = jnp.mean(jnp.square(hv - mu[None, :]), axis=0)
    hn = g_ref[...] * (hv - mu[None, :]) * lax.rsqrt(var + 1e-5)[None, :]
    out_ref[...] = jnp.maximum(hn + b_ref[...], 0.0)


def _tc_post(agg2, den3, hs, gamma, beta):
    return pl.pallas_call(
        _tc_post_body,
        out_shape=jax.ShapeDtypeStruct((NN, HD), jnp.float32),
    )(agg2, den3, hs, gamma, beta)


def kernel(x, edge_index, edge_curvature, params):
    src = edge_index[0]
    dst = edge_index[1]
    pad = EPAD - EE
    srcp = jnp.pad(src, (0, pad)).reshape(NW, CHUNKS, CH)
    dstp = jnp.pad(dst, (0, pad)).reshape(NW, CHUNKS, CH)
    curvp = jnp.pad(edge_curvature, (0, pad))
    valid = jnp.arange(EPAD, dtype=jnp.int32) < EE
    masks = {
        'positive': ((curvp > 0) & valid).astype(jnp.float32).reshape(
            NW, CHUNKS, CH),
        'negative': ((curvp < 0) & valid).astype(jnp.float32).reshape(
            NW, CHUNKS, CH),
        'both': valid.astype(jnp.float32).reshape(NW, CHUNKS, CH),
    }

    h0 = _tc_in(x, params['W_in'], params['b_in'])
    outs = []
    for ct in ('positive', 'negative', 'both'):
        h = h0
        for i in range(3):
            p = ct + '_' + str(i)
            a2 = jnp.stack([params['asrc_' + p], params['adst_' + p]], axis=1)
            m, aa, hs = _tc_pre(h, params['W_' + p], params['Wself_' + p],
                                params['b_' + p], a2)
            avt = jnp.pad(aa, ((0, NPAD - NN), (0, 0))).T
            agg2, den = _sc_edge(m, avt, srcp, dstp, masks[ct])
            den3 = den.reshape(NCORE, DRP * HD, 1)
            h = _tc_post(agg2, den3, hs, params['bn_gamma_' + str(i)],
                         params['bn_beta_' + str(i)])
            outs.append(h)
    return jnp.stack(outs, axis=0)


# speedup vs baseline: 9.6859x; 1.2105x over previous
import functools

import jax
import jax.numpy as jnp
from jax import lax
from jax.experimental import pallas as pl
from jax.experimental.pallas import tpu as pltpu
from jax.experimental.pallas import tpu_sc as plsc

NN = 10000
EE = 320000
HD = 128
NCORE = 2
NSUB = 16
NW = NCORE * NSUB
CH = 64
BLK = 4
CHUNKS = 160
BLOCKS = CHUNKS // BLK
EPT = CHUNKS * CH
EPAD = EPT * NW
NPAD = 10112
DR = 80
DRP = 80
DRT = 8
NPT = NPAD // NSUB


def _sc_body(mtab, avt, srcs, dsts, masks, agg_out, den_out,
             src_b, dst_b, msk_b, as_v, ad_v, den_v, exc_v,
             rows0_v, rows1_v, idx_v, agg_sh, den_sh, sem0, sem1):
    c = lax.axis_index("c")
    s = lax.axis_index("s")
    wid = c * NSUB + s

    pltpu.sync_copy(avt.at[0], as_v)
    pltpu.sync_copy(avt.at[1], ad_v)

    zeros16 = jnp.zeros((16,), jnp.float32)

    def zden(i, carry):
        for k in range(HD // 16):
            den_v[i, pl.ds(k * 16, 16)] = zeros16
        return carry

    lax.fori_loop(0, DR, zden, 0)

    def zrow(i, carry):
        for k in range(HD // 16):
            rows0_v[i, pl.ds(k * 16, 16)] = zeros16
        return carry

    lax.fori_loop(0, CH, zrow, 0)

    for g in range(DR // 16):
        idx_v[0, pl.ds(g * 16, 16)] = (
            lax.iota(jnp.int32, 16) + jnp.full((16,), g * 16, jnp.int32))

    for k in range(NPT // CH):
        pltpu.sync_copy(rows0_v, agg_sh.at[pl.ds(s * NPT + k * CH, CH)])
    rem = NPT - (NPT // CH) * CH
    if rem:
        pltpu.sync_copy(rows0_v.at[pl.ds(0, rem)],
                        agg_sh.at[pl.ds(s * NPT + (NPT // CH) * CH, rem)])

    @pl.when(s < DRP // DRT)
    def _zero_den():
        pltpu.sync_copy(rows0_v.at[pl.ds(0, DRT)],
                        den_sh.at[pl.ds(s * DRT, DRT)])

    plsc.subcore_barrier()

    rows = (rows0_v, rows1_v)
    sems = (sem0, sem1)

    def block(b, carry):
        pltpu.sync_copy(srcs.at[wid, pl.ds(b * BLK, BLK)], src_b)
        pltpu.sync_copy(dsts.at[wid, pl.ds(b * BLK, BLK)], dst_b)
        pltpu.sync_copy(masks.at[wid, pl.ds(b * BLK, BLK)], msk_b)
        cps = [pltpu.async_copy(mtab.at[src_b.at[0]], rows[0], sems[0]),
               pltpu.async_copy(mtab.at[src_b.at[1]], rows[1], sems[1])]
        for q in range(BLK):
            cur = q % 2
            for j in range(CH // 16):
                sid = src_b[q, pl.ds(j * 16, 16)]
                did = dst_b[q, pl.ds(j * 16, 16)]
                a = (plsc.load_gather(as_v, [sid])
                     + plsc.load_gather(ad_v, [did]))
                a = jnp.maximum(a, a * 0.2)
                a = jnp.minimum(jnp.maximum(a, -30.0), 30.0)
                ex = msk_b[q, pl.ds(j * 16, 16)] * jnp.exp(a)
                exc_v[pl.ds(j * 16, 16)] = ex
                plsc.addupdate_scatter(
                    den_v,
                    [lax.shift_right_logical(did, 7),
                     lax.bitwise_and(did, jnp.full((16,), 127, jnp.int32))],
                    ex)
            cps[cur].wait()
            rv = rows[cur]

            def scale_i(i, carry2, rv=rv):
                cb = plsc.load_gather(exc_v, [jnp.full((16,), i, jnp.int32)])
                for k in range(HD // 16):
                    rv[i, pl.ds(k * 16, 16)] = rv[i, pl.ds(k * 16, 16)] * cb
                return carry2

            lax.fori_loop(0, CH, scale_i, 0)
            pltpu.sync_copy(rv, agg_sh.at[dst_b.at[q]], add=True)
            if q + 2 < BLK:
                cps[cur] = pltpu.async_copy(
                    mtab.at[src_b.at[q + 2]], rv, sems[cur])
        return carry

    lax.fori_loop(0, BLOCKS, block, 0)

    pltpu.sync_copy(den_v, den_sh.at[idx_v.at[0]], add=True)
    plsc.subcore_barrier()

    pltpu.sync_copy(agg_sh.at[pl.ds(s * NPT, NPT)],
                    agg_out.at[pl.ds(c * NPAD + s * NPT, NPT)])

    @pl.when(s < DRP // DRT)
    def _write_den():
        pltpu.sync_copy(den_sh.at[pl.ds(s * DRT, DRT)],
                        den_out.at[c, pl.ds(s * DRT, DRT)])


_sc_edge = functools.partial(
    pl.kernel,
    out_type=(
        jax.ShapeDtypeStruct((NCORE * NPAD, HD), jnp.float32),
        jax.ShapeDtypeStruct((NCORE, DRP, HD), jnp.float32),
    ),
    mesh=plsc.VectorSubcoreMesh(core_axis_name="c", subcore_axis_name="s",
                                num_cores=NCORE, num_subcores=NSUB),
    scratch_types=[
        pltpu.VMEM((BLK, CH), jnp.int32),
        pltpu.VMEM((BLK, CH), jnp.int32),
        pltpu.VMEM((BLK, CH), jnp.float32),
        pltpu.VMEM((NPAD,), jnp.float32),
        pltpu.VMEM((NPAD,), jnp.float32),
        pltpu.VMEM((DR, HD), jnp.float32),
        pltpu.VMEM((CH,), jnp.float32),
        pltpu.VMEM((CH, HD), jnp.float32),
        pltpu.VMEM((CH, HD), jnp.float32),
        pltpu.VMEM((1, DR), jnp.int32),
        pltpu.VMEM_SHARED((NPAD, HD), jnp.float32),
        pltpu.VMEM_SHARED((DRP, HD), jnp.float32),
        pltpu.SemaphoreType.DMA,
        pltpu.SemaphoreType.DMA,
    ],
    compiler_params=pltpu.CompilerParams(needs_layout_passes=False),
)(_sc_body)


def _tc_in_body(x_ref, w_ref, b_ref, out_ref):
    h = jnp.dot(x_ref[...], w_ref[...], preferred_element_type=jnp.float32)
    out_ref[...] = jnp.maximum(h + b_ref[...], 0.0)


def _tc_in(x, w, b):
    return pl.pallas_call(
        _tc_in_body,
        out_shape=jax.ShapeDtypeStruct((NN, HD), jnp.float32),
    )(x, w, b)


def _tc_pre_body(h_ref, w_ref, wself_ref, b_ref, a2_ref,
                 m_ref, aa_ref, hs_ref):
    h = h_ref[...]
    m = jnp.dot(h, w_ref[...], preferred_element_type=jnp.float32)
    m_ref[...] = m
    aa_ref[...] = jnp.dot(m, a2_ref[...], preferred_element_type=jnp.float32)
    hs_ref[...] = (jnp.dot(h, wself_ref[...],
                           preferred_element_type=jnp.float32) + b_ref[...])


def _tc_pre(h, w, wself, b, a2):
    return pl.pallas_call(
        _tc_pre_body,
        out_shape=(
            jax.ShapeDtypeStruct((NN, HD), jnp.float32),
            jax.ShapeDtypeStruct((NN, 2), jnp.float32),
            jax.ShapeDtypeStruct((NN, HD), jnp.float32),
        ),
    )(h, w, wself, b, a2)


def _tc_post_body(agg2_ref, den3_ref, hs_ref, g_ref, b_ref, out_ref):
    agg = agg2_ref[pl.ds(0, NN), :] + agg2_ref[pl.ds(NPAD, NN), :]
    den = (den3_ref[0] + den3_ref[1])[:NN, :]
    hv = agg / (den + 1e-16) + hs_ref[...]
    mu = jnp.mean(hv, axis=0)
    var = jnp.mean(jnp.square(hv - mu[None, :]), axis=0)
    hn = g_ref[...] * (hv - mu[None, :]) * lax.rsqrt(var + 1e-5)[None, :]
    out_ref[...] = jnp.maximum(hn + b_ref[...], 0.0)


def _tc_post(agg2, den3, hs, gamma, beta):
    return pl.pallas_call(
        _tc_post_body,
        out_shape=jax.ShapeDtypeStruct((NN, HD), jnp.float32),
    )(agg2, den3, hs, gamma, beta)


def kernel(x, edge_index, edge_curvature, params):
    src = edge_index[0]
    dst = edge_index[1]
    pad = EPAD - EE
    srcp = jnp.pad(src, (0, pad)).reshape(NW, CHUNKS, CH)
    dstp = jnp.pad(dst, (0, pad)).reshape(NW, CHUNKS, CH)
    curvp = jnp.pad(edge_curvature, (0, pad))
    valid = jnp.arange(EPAD, dtype=jnp.int32) < EE
    masks = {
        'positive': ((curvp > 0) & valid).astype(jnp.float32).reshape(
            NW, CHUNKS, CH),
        'negative': ((curvp < 0) & valid).astype(jnp.float32).reshape(
            NW, CHUNKS, CH),
        'both': valid.astype(jnp.float32).reshape(NW, CHUNKS, CH),
    }

    h0 = _tc_in(x, params['W_in'], params['b_in'])
    outs = []
    for ct in ('positive', 'negative', 'both'):
        h = h0
        for i in range(3):
            p = ct + '_' + str(i)
            a2 = jnp.stack([params['asrc_' + p], params['adst_' + p]], axis=1)
            m, aa, hs = _tc_pre(h, params['W_' + p], params['Wself_' + p],
                                params['b_' + p], a2)
            avt = jnp.pad(aa, ((0, NPAD - NN), (0, 0))).T
            agg2, den = _sc_edge(m, avt, srcp, dstp, masks[ct])
            den3 = den.reshape(NCORE, DRP * HD, 1)
            h = _tc_post(agg2, den3, hs, params['bn_gamma_' + str(i)],
                         params['bn_beta_' + str(i)])
            outs.append(h)
    return jnp.stack(outs, axis=0)


# speedup vs baseline: 10.4586x; 1.0798x over previous
import functools

import jax
import jax.numpy as jnp
from jax import lax
from jax.experimental import pallas as pl
from jax.experimental.pallas import tpu as pltpu
from jax.experimental.pallas import tpu_sc as plsc

NN = 10000
EE = 320000
HD = 128
NCORE = 2
NSUB = 16
NW = NCORE * NSUB
CH = 64
BLK = 4
CHUNKS = 160
BLOCKS = CHUNKS // BLK
EPT = CHUNKS * CH
EPAD = EPT * NW
NPAD = 10112
DR = 80
DRP = 80
DRT = 8
NPT = NPAD // NSUB


def _sc_body(mtab, avt, srcs, dsts, masks, agg_out, den_out,
             src_b, dst_b, msk_b, as_v, ad_v, den_v, exc_v,
             rows0_v, rows1_v, idx_v, agg_sh, den_sh,
             sem0, sem1, ssem0, ssem1):
    c = lax.axis_index("c")
    s = lax.axis_index("s")
    wid = c * NSUB + s

    pltpu.sync_copy(avt.at[0], as_v)
    pltpu.sync_copy(avt.at[1], ad_v)

    zeros16 = jnp.zeros((16,), jnp.float32)

    def zden(i, carry):
        for k in range(HD // 16):
            den_v[i, pl.ds(k * 16, 16)] = zeros16
        return carry

    lax.fori_loop(0, DR, zden, 0)

    def zrow(i, carry):
        for k in range(HD // 16):
            rows0_v[i, pl.ds(k * 16, 16)] = zeros16
        return carry

    lax.fori_loop(0, CH, zrow, 0)

    for g in range(DR // 16):
        idx_v[0, pl.ds(g * 16, 16)] = (
            lax.iota(jnp.int32, 16) + jnp.full((16,), g * 16, jnp.int32))

    for k in range(NPT // CH):
        pltpu.sync_copy(rows0_v, agg_sh.at[pl.ds(s * NPT + k * CH, CH)])
    rem = NPT - (NPT // CH) * CH
    if rem:
        pltpu.sync_copy(rows0_v.at[pl.ds(0, rem)],
                        agg_sh.at[pl.ds(s * NPT + (NPT // CH) * CH, rem)])

    @pl.when(s < DRP // DRT)
    def _zero_den():
        pltpu.sync_copy(rows0_v.at[pl.ds(0, DRT)],
                        den_sh.at[pl.ds(s * DRT, DRT)])

    plsc.subcore_barrier()

    rows = (rows0_v, rows1_v)
    sems = (sem0, sem1)
    ssems = (ssem0, ssem1)

    def block(b, carry):
        pltpu.sync_copy(srcs.at[wid, pl.ds(b * BLK, BLK)], src_b)
        pltpu.sync_copy(dsts.at[wid, pl.ds(b * BLK, BLK)], dst_b)
        pltpu.sync_copy(masks.at[wid, pl.ds(b * BLK, BLK)], msk_b)
        cps = [pltpu.async_copy(mtab.at[src_b.at[0]], rows[0], sems[0]),
               None]
        scps = [None, None]
        for q in range(BLK):
            cur = q % 2
            prv = 1 - cur
            for j in range(CH // 16):
                sid = src_b[q, pl.ds(j * 16, 16)]
                did = dst_b[q, pl.ds(j * 16, 16)]
                a = (plsc.load_gather(as_v, [sid])
                     + plsc.load_gather(ad_v, [did]))
                a = jnp.maximum(a, a * 0.2)
                a = jnp.minimum(jnp.maximum(a, -30.0), 30.0)
                ex = msk_b[q, pl.ds(j * 16, 16)] * jnp.exp(a)
                exc_v[pl.ds(j * 16, 16)] = ex
                plsc.addupdate_scatter(
                    den_v,
                    [lax.shift_right_logical(did, 7),
                     lax.bitwise_and(did, jnp.full((16,), 127, jnp.int32))],
                    ex)
            if scps[prv] is not None:
                scps[prv].wait()
            if q + 1 < BLK:
                cps[prv] = pltpu.async_copy(
                    mtab.at[src_b.at[q + 1]], rows[prv], sems[prv])
            cps[cur].wait()
            rv = rows[cur]

            def scale_i(i, carry2, rv=rv):
                for u in range(4):
                    cb = plsc.load_gather(
                        exc_v, [jnp.full((16,), u, jnp.int32) + i * 4])
                    for k in range(HD // 16):
                        rv[i * 4 + u, pl.ds(k * 16, 16)] = (
                            rv[i * 4 + u, pl.ds(k * 16, 16)] * cb)
                return carry2

            lax.fori_loop(0, CH // 4, scale_i, 0)
            scps[cur] = pltpu.async_copy(
                rv, agg_sh.at[dst_b.at[q]], ssems[cur], add=True)
        scps[(BLK - 1) % 2].wait()
        return carry

    lax.fori_loop(0, BLOCKS, block, 0)

    pltpu.sync_copy(den_v, den_sh.at[idx_v.at[0]], add=True)
    plsc.subcore_barrier()

    pltpu.sync_copy(agg_sh.at[pl.ds(s * NPT, NPT)],
                    agg_out.at[pl.ds(c * NPAD + s * NPT, NPT)])

    @pl.when(s < DRP // DRT)
    def _write_den():
        pltpu.sync_copy(den_sh.at[pl.ds(s * DRT, DRT)],
                        den_out.at[c, pl.ds(s * DRT, DRT)])


_sc_edge = functools.partial(
    pl.kernel,
    out_type=(
        jax.ShapeDtypeStruct((NCORE * NPAD, HD), jnp.float32),
        jax.ShapeDtypeStruct((NCORE, DRP, HD), jnp.float32),
    ),
    mesh=plsc.VectorSubcoreMesh(core_axis_name="c", subcore_axis_name="s",
                                num_cores=NCORE, num_subcores=NSUB),
    scratch_types=[
        pltpu.VMEM((BLK, CH), jnp.int32),
        pltpu.VMEM((BLK, CH), jnp.int32),
        pltpu.VMEM((BLK, CH), jnp.float32),
        pltpu.VMEM((NPAD,), jnp.float32),
        pltpu.VMEM((NPAD,), jnp.float32),
        pltpu.VMEM((DR, HD), jnp.float32),
        pltpu.VMEM((CH,), jnp.float32),
        pltpu.VMEM((CH, HD), jnp.float32),
        pltpu.VMEM((CH, HD), jnp.float32),
        pltpu.VMEM((1, DR), jnp.int32),
        pltpu.VMEM_SHARED((NPAD, HD), jnp.float32),
        pltpu.VMEM_SHARED((DRP, HD), jnp.float32),
        pltpu.SemaphoreType.DMA,
        pltpu.SemaphoreType.DMA,
        pltpu.SemaphoreType.DMA,
        pltpu.SemaphoreType.DMA,
    ],
    compiler_params=pltpu.CompilerParams(needs_layout_passes=False),
)(_sc_body)


def _tc_in_body(x_ref, w_ref, b_ref, out_ref):
    h = jnp.dot(x_ref[...], w_ref[...], preferred_element_type=jnp.float32)
    out_ref[...] = jnp.maximum(h + b_ref[...], 0.0)


def _tc_in(x, w, b):
    return pl.pallas_call(
        _tc_in_body,
        out_shape=jax.ShapeDtypeStruct((NN, HD), jnp.float32),
    )(x, w, b)


def _tc_pre_body(h_ref, w_ref, wself_ref, b_ref, a2_ref,
                 m_ref, aa_ref, hs_ref):
    h = h_ref[...]
    m = jnp.dot(h, w_ref[...], preferred_element_type=jnp.float32)
    m_ref[...] = m
    aa_ref[...] = jnp.dot(m, a2_ref[...], preferred_element_type=jnp.float32)
    hs_ref[...] = (jnp.dot(h, wself_ref[...],
                           preferred_element_type=jnp.float32) + b_ref[...])


def _tc_pre(h, w, wself, b, a2):
    return pl.pallas_call(
        _tc_pre_body,
        out_shape=(
            jax.ShapeDtypeStruct((NN, HD), jnp.float32),
            jax.ShapeDtypeStruct((NN, 2), jnp.float32),
            jax.ShapeDtypeStruct((NN, HD), jnp.float32),
        ),
    )(h, w, wself, b, a2)


def _tc_post_body(agg2_ref, den3_ref, hs_ref, g_ref, b_ref, out_ref):
    agg = agg2_ref[pl.ds(0, NN), :] + agg2_ref[pl.ds(NPAD, NN), :]
    den = (den3_ref[0] + den3_ref[1])[:NN, :]
    hv = agg / (den + 1e-16) + hs_ref[...]
    mu = jnp.mean(hv, axis=0)
    var = jnp.mean(jnp.square(hv - mu[None, :]), axis=0)
    hn = g_ref[...] * (hv - mu[None, :]) * lax.rsqrt(var + 1e-5)[None, :]
    out_ref[...] = jnp.maximum(hn + b_ref[...], 0.0)


def _tc_post(agg2, den3, hs, gamma, beta):
    return pl.pallas_call(
        _tc_post_body,
        out_shape=jax.ShapeDtypeStruct((NN, HD), jnp.float32),
    )(agg2, den3, hs, gamma, beta)


def kernel(x, edge_index, edge_curvature, params):
    src = edge_index[0]
    dst = edge_index[1]
    pad = EPAD - EE
    srcp = jnp.pad(src, (0, pad)).reshape(NW, CHUNKS, CH)
    dstp = jnp.pad(dst, (0, pad)).reshape(NW, CHUNKS, CH)
    curvp = jnp.pad(edge_curvature, (0, pad))
    valid = jnp.arange(EPAD, dtype=jnp.int32) < EE
    masks = {
        'positive': ((curvp > 0) & valid).astype(jnp.float32).reshape(
            NW, CHUNKS, CH),
        'negative': ((curvp < 0) & valid).astype(jnp.float32).reshape(
            NW, CHUNKS, CH),
        'both': valid.astype(jnp.float32).reshape(NW, CHUNKS, CH),
    }

    h0 = _tc_in(x, params['W_in'], params['b_in'])
    outs = []
    for ct in ('positive', 'negative', 'both'):
        h = h0
        for i in range(3):
            p = ct + '_' + str(i)
            a2 = jnp.stack([params['asrc_' + p], params['adst_' + p]], axis=1)
            m, aa, hs = _tc_pre(h, params['W_' + p], params['Wself_' + p],
                                params['b_' + p], a2)
            avt = jnp.pad(aa, ((0, NPAD - NN), (0, 0))).T
            agg2, den = _sc_edge(m, avt, srcp, dstp, masks[ct])
            den3 = den.reshape(NCORE, DRP * HD, 1)
            h = _tc_post(agg2, den3, hs, params['bn_gamma_' + str(i)],
                         params['bn_beta_' + str(i)])
            outs.append(h)
    return jnp.stack(outs, axis=0)


# speedup vs baseline: 11.0725x; 1.0587x over previous
import functools

import jax
import jax.numpy as jnp
from jax import lax
from jax.experimental import pallas as pl
from jax.experimental.pallas import tpu as pltpu
from jax.experimental.pallas import tpu_sc as plsc

NN = 10000
EE = 320000
HD = 128
NCORE = 2
NSUB = 16
NW = NCORE * NSUB
CH = 64
BLK = 4
CHUNKS = 160
BLOCKS = CHUNKS // BLK
EPT = CHUNKS * CH
EPAD = EPT * NW
NPAD = 10112
DR = 80
DRP = 80
DRT = 8
NPT = NPAD // NSUB


def _sc_body(mtab, avt, sd, agg_out, den_out,
             sdb0, sdb1, as_v, ad_v, den_v, exc_v,
             rows0_v, rows1_v, idx_v, agg_sh, den_sh,
             sem0, sem1, ssem0, ssem1, stsem0, stsem1):
    c = lax.axis_index("c")
    s = lax.axis_index("s")
    wid = c * NSUB + s

    pltpu.sync_copy(avt.at[0], as_v)
    pltpu.sync_copy(avt.at[1], ad_v)

    zeros16 = jnp.zeros((16,), jnp.float32)

    def zden(i, carry):
        for k in range(HD // 16):
            den_v[i, pl.ds(k * 16, 16)] = zeros16
        return carry

    lax.fori_loop(0, DR, zden, 0)

    def zrow(i, carry):
        for k in range(HD // 16):
            rows0_v[i, pl.ds(k * 16, 16)] = zeros16
        return carry

    lax.fori_loop(0, CH, zrow, 0)

    for g in range(DR // 16):
        idx_v[0, pl.ds(g * 16, 16)] = (
            lax.iota(jnp.int32, 16) + jnp.full((16,), g * 16, jnp.int32))

    for k in range(NPT // CH):
        pltpu.sync_copy(rows0_v, agg_sh.at[pl.ds(s * NPT + k * CH, CH)])
    rem = NPT - (NPT // CH) * CH
    if rem:
        pltpu.sync_copy(rows0_v.at[pl.ds(0, rem)],
                        agg_sh.at[pl.ds(s * NPT + (NPT // CH) * CH, rem)])

    @pl.when(s < DRP // DRT)
    def _zero_den():
        pltpu.sync_copy(rows0_v.at[pl.ds(0, DRT)],
                        den_sh.at[pl.ds(s * DRT, DRT)])

    plsc.subcore_barrier()

    rows = (rows0_v, rows1_v)
    sems = (sem0, sem1)
    ssems = (ssem0, ssem1)
    BR = BLK * 2

    def do_block(sdb):
        cps = [pltpu.async_copy(mtab.at[sdb.at[0]], rows[0], sems[0]),
               None]
        scps = [None, None]
        for q in range(BLK):
            cur = q % 2
            prv = 1 - cur
            for j in range(CH // 16):
                sid = sdb[2 * q, pl.ds(j * 16, 16)]
                did = sdb[2 * q + 1, pl.ds(j * 16, 16)]
                a = (plsc.load_gather(as_v, [sid])
                     + plsc.load_gather(ad_v, [did]))
                a = jnp.maximum(a, a * 0.2)
                a = jnp.minimum(jnp.maximum(a, -30.0), 30.0)
                ex = jnp.exp(a)
                exc_v[pl.ds(j * 16, 16)] = ex
                plsc.addupdate_scatter(
                    den_v,
                    [lax.shift_right_logical(did, 7),
                     lax.bitwise_and(did, jnp.full((16,), 127, jnp.int32))],
                    ex)
            if scps[prv] is not None:
                scps[prv].wait()
            if q + 1 < BLK:
                cps[prv] = pltpu.async_copy(
                    mtab.at[sdb.at[2 * (q + 1)]], rows[prv], sems[prv])
            cps[cur].wait()
            rv = rows[cur]

            def scale_i(i, carry2, rv=rv):
                for u in range(4):
                    cb = plsc.load_gather(
                        exc_v, [jnp.full((16,), u, jnp.int32) + i * 4])
                    for k in range(HD // 16):
                        rv[i * 4 + u, pl.ds(k * 16, 16)] = (
                            rv[i * 4 + u, pl.ds(k * 16, 16)] * cb)
                return carry2

            lax.fori_loop(0, CH // 4, scale_i, 0)
            scps[cur] = pltpu.async_copy(
                rv, agg_sh.at[sdb.at[2 * q + 1]], ssems[cur], add=True)
        scps[(BLK - 1) % 2].wait()

    pltpu.async_copy(sd.at[wid, pl.ds(0, BR)], sdb0, stsem0)

    def dblock(t, carry):
        b0 = t * 2
        pltpu.async_copy(sd.at[wid, pl.ds((b0 + 1) * BR, BR)], sdb1, stsem1)
        pltpu.make_async_copy(sd.at[wid, pl.ds(0, BR)], sdb0, stsem0).wait()
        do_block(sdb0)
        pltpu.async_copy(sd.at[wid, pl.ds((b0 + 2) * BR, BR)], sdb0, stsem0)
        pltpu.make_async_copy(sd.at[wid, pl.ds(0, BR)], sdb1, stsem1).wait()
        do_block(sdb1)
        return carry

    lax.fori_loop(0, BLOCKS // 2, dblock, 0)
    pltpu.make_async_copy(sd.at[wid, pl.ds(0, BR)], sdb0, stsem0).wait()

    pltpu.sync_copy(den_v, den_sh.at[idx_v.at[0]], add=True)
    plsc.subcore_barrier()

    pltpu.sync_copy(agg_sh.at[pl.ds(s * NPT, NPT)],
                    agg_out.at[pl.ds(c * NPAD + s * NPT, NPT)])

    @pl.when(s < DRP // DRT)
    def _write_den():
        pltpu.sync_copy(den_sh.at[pl.ds(s * DRT, DRT)],
                        den_out.at[c, pl.ds(s * DRT, DRT)])


_sc_edge = functools.partial(
    pl.kernel,
    out_type=(
        jax.ShapeDtypeStruct((NCORE * NPAD, HD), jnp.float32),
        jax.ShapeDtypeStruct((NCORE, DRP, HD), jnp.float32),
    ),
    mesh=plsc.VectorSubcoreMesh(core_axis_name="c", subcore_axis_name="s",
                                num_cores=NCORE, num_subcores=NSUB),
    scratch_types=[
        pltpu.VMEM((BLK * 2, CH), jnp.int32),
        pltpu.VMEM((BLK * 2, CH), jnp.int32),
        pltpu.VMEM((NPAD,), jnp.float32),
        pltpu.VMEM((NPAD,), jnp.float32),
        pltpu.VMEM((DR, HD), jnp.float32),
        pltpu.VMEM((CH,), jnp.float32),
        pltpu.VMEM((CH, HD), jnp.float32),
        pltpu.VMEM((CH, HD), jnp.float32),
        pltpu.VMEM((1, DR), jnp.int32),
        pltpu.VMEM_SHARED((NPAD, HD), jnp.float32),
        pltpu.VMEM_SHARED((DRP, HD), jnp.float32),
        pltpu.SemaphoreType.DMA,
        pltpu.SemaphoreType.DMA,
        pltpu.SemaphoreType.DMA,
        pltpu.SemaphoreType.DMA,
        pltpu.SemaphoreType.DMA,
        pltpu.SemaphoreType.DMA,
    ],
    compiler_params=pltpu.CompilerParams(needs_layout_passes=False),
)(_sc_body)


def _tc_in_body(x_ref, w_ref, b_ref, out_ref):
    h = jnp.dot(x_ref[...], w_ref[...], preferred_element_type=jnp.float32)
    out_ref[...] = jnp.maximum(h + b_ref[...], 0.0)


def _tc_in(x, w, b):
    return pl.pallas_call(
        _tc_in_body,
        out_shape=jax.ShapeDtypeStruct((NN, HD), jnp.float32),
    )(x, w, b)


def _tc_pre_body(h_ref, w_ref, wself_ref, b_ref, a2_ref,
                 m_ref, aa_ref, hs_ref):
    h = h_ref[...]
    m = jnp.dot(h, w_ref[...], preferred_element_type=jnp.float32)
    m_ref[...] = m
    aa_ref[...] = jnp.dot(m, a2_ref[...], preferred_element_type=jnp.float32)
    hs_ref[...] = (jnp.dot(h, wself_ref[...],
                           preferred_element_type=jnp.float32) + b_ref[...])


def _tc_pre(h, w, wself, b, a2):
    return pl.pallas_call(
        _tc_pre_body,
        out_shape=(
            jax.ShapeDtypeStruct((NN, HD), jnp.float32),
            jax.ShapeDtypeStruct((NN, 2), jnp.float32),
            jax.ShapeDtypeStruct((NN, HD), jnp.float32),
        ),
    )(h, w, wself, b, a2)


def _tc_post_body(agg2_ref, den3_ref, hs_ref, g_ref, b_ref, out_ref):
    agg = agg2_ref[pl.ds(0, NN), :] + agg2_ref[pl.ds(NPAD, NN), :]
    den = (den3_ref[0] + den3_ref[1])[:NN, :]
    hv = agg / (den + 1e-16) + hs_ref[...]
    mu = jnp.mean(hv, axis=0)
    var = jnp.mean(jnp.square(hv - mu[None, :]), axis=0)
    hn = g_ref[...] * (hv - mu[None, :]) * lax.rsqrt(var + 1e-5)[None, :]
    out_ref[...] = jnp.maximum(hn + b_ref[...], 0.0)


def _tc_post(agg2, den3, hs, gamma, beta):
    return pl.pallas_call(
        _tc_post_body,
        out_shape=jax.ShapeDtypeStruct((NN, HD), jnp.float32),
    )(agg2, den3, hs, gamma, beta)


def kernel(x, edge_index, edge_curvature, params):
    src = edge_index[0]
    dst = edge_index[1]
    pad = EPAD - EE
    srcp = jnp.pad(src, (0, pad)).reshape(NW, CHUNKS, CH)
    dstp = jnp.pad(dst, (0, pad))
    curvp = jnp.pad(edge_curvature, (0, pad))
    valid = jnp.arange(EPAD, dtype=jnp.int32) < EE
    dummy = NN + (jnp.arange(EPAD, dtype=jnp.int32) % (NPAD - NN))

    def make_sd(active):
        d = jnp.where(active & valid, dstp, dummy).reshape(NW, CHUNKS, CH)
        s2 = jnp.stack([srcp, d], axis=2).reshape(NW, CHUNKS * 2, CH)
        return jnp.pad(s2, ((0, 0), (0, BLK * 2), (0, 0)))

    sds = {
        'positive': make_sd(curvp > 0),
        'negative': make_sd(curvp < 0),
        'both': make_sd(jnp.ones((EPAD,), jnp.bool_)),
    }

    h0 = _tc_in(x, params['W_in'], params['b_in'])
    outs = []
    for ct in ('positive', 'negative', 'both'):
        h = h0
        for i in range(3):
            p = ct + '_' + str(i)
            a2 = jnp.stack([params['asrc_' + p], params['adst_' + p]], axis=1)
            m, aa, hs = _tc_pre(h, params['W_' + p], params['Wself_' + p],
                                params['b_' + p], a2)
            avt = jnp.pad(aa, ((0, NPAD - NN), (0, 0))).T
            agg2, den = _sc_edge(m, avt, sds[ct])
            den3 = den.reshape(NCORE, DRP * HD, 1)
            h = _tc_post(agg2, den3, hs, params['bn_gamma_' + str(i)],
                         params['bn_beta_' + str(i)])
            outs.append(h)
    return jnp.stack(outs, axis=0)
